# baseline (speedup 1.0000x reference)
import functools

import jax
import jax.numpy as jnp
from jax import lax
from jax.experimental import pallas as pl
from jax.experimental.pallas import tpu as pltpu
from jax.experimental.pallas import tpu_sc as plsc


def _sc_gather(emb, t):
    info = plsc.get_sparse_core_info()
    nw = info.num_cores * info.num_subcores
    B = t.shape[0]
    D = emb.shape[1]
    b_per_w = B // nw
    mesh = plsc.VectorSubcoreMesh(core_axis_name="c", subcore_axis_name="s")

    @functools.partial(
        pl.kernel,
        mesh=mesh,
        out_type=jax.ShapeDtypeStruct((B, D), jnp.float32),
        scratch_types=[
            pltpu.VMEM((b_per_w,), jnp.int32),
            pltpu.VMEM((b_per_w, D), jnp.float32),
            pltpu.SemaphoreType.DMA,
        ],
    )
    def gather(emb_hbm, t_hbm, out_hbm, idx_v, rows_v, sem):
        wid = lax.axis_index("s") * info.num_cores + lax.axis_index("c")
        base = wid * b_per_w
        pltpu.sync_copy(t_hbm.at[pl.ds(base, b_per_w)], idx_v)
        pltpu.async_copy(emb_hbm.at[idx_v], rows_v, sem).wait()
        pltpu.sync_copy(rows_v, out_hbm.at[pl.ds(base, b_per_w)])

    return gather(emb, t)


def _fused_body(e_ref, w_ref, b_ref, x_ref, out_ref, te_ref):
    @pl.when(pl.program_id(1) == 0)
    def _compute_te():
        te_ref[...] = (
            lax.dot_general(
                e_ref[...], w_ref[...],
                (((1,), (1,)), ((), ())),
                preferred_element_type=jnp.float32,
            )
            + b_ref[...]
        )

    out_ref[...] = x_ref[...] + te_ref[...][None]


def _fused_add(e, x, W, b, *, chunk=512):
    B2, H, T, N = x.shape
    bh = B2 * H
    s = T // chunk
    xr = x.reshape(bh, T, N)
    out = pl.pallas_call(
        _fused_body,
        grid=(s, bh),
        in_specs=[
            pl.BlockSpec((chunk, N), lambda i, j: (i, 0)),
            pl.BlockSpec((N, N), lambda i, j: (0, 0)),
            pl.BlockSpec((1, N), lambda i, j: (0, 0)),
            pl.BlockSpec((1, chunk, N), lambda i, j: (j, i, 0)),
        ],
        out_specs=pl.BlockSpec((1, chunk, N), lambda i, j: (j, i, 0)),
        out_shape=jax.ShapeDtypeStruct((bh, T, N), jnp.float32),
        scratch_shapes=[pltpu.VMEM((chunk, N), jnp.float32)],
    )(e, W, b.reshape(1, N), xr)
    return out.reshape(B2, H, T, N)


def kernel(x, t, emb, W, b):
    e = _sc_gather(emb, t)
    return _fused_add(e, x, W, b)


# speedup vs baseline: 1.0831x; 1.0831x over previous
import functools

import jax
import jax.numpy as jnp
from jax import lax
from jax.experimental import pallas as pl
from jax.experimental.pallas import tpu as pltpu
from jax.experimental.pallas import tpu_sc as plsc


def _sc_gather(emb, t):
    info = plsc.get_sparse_core_info()
    nw = info.num_cores * info.num_subcores
    B = t.shape[0]
    D = emb.shape[1]
    b_per_w = B // nw
    mesh = plsc.VectorSubcoreMesh(core_axis_name="c", subcore_axis_name="s")

    @functools.partial(
        pl.kernel,
        mesh=mesh,
        out_type=jax.ShapeDtypeStruct((B, D), jnp.float32),
        scratch_types=[
            pltpu.VMEM((b_per_w,), jnp.int32),
            pltpu.VMEM((b_per_w, D), jnp.float32),
            pltpu.SemaphoreType.DMA,
        ],
    )
    def gather(emb_hbm, t_hbm, out_hbm, idx_v, rows_v, sem):
        wid = lax.axis_index("s") * info.num_cores + lax.axis_index("c")
        base = wid * b_per_w
        pltpu.sync_copy(t_hbm.at[pl.ds(base, b_per_w)], idx_v)
        pltpu.async_copy(emb_hbm.at[idx_v], rows_v, sem).wait()
        pltpu.sync_copy(rows_v, out_hbm.at[pl.ds(base, b_per_w)])

    return gather(emb, t)


def _fused_body(e_ref, w_ref, b_ref, x_ref, out_ref, te_ref):
    @pl.when(pl.program_id(1) == 0)
    def _compute_te():
        te_ref[...] = (
            lax.dot_general(
                e_ref[...].astype(jnp.bfloat16), w_ref[...],
                (((1,), (1,)), ((), ())),
                preferred_element_type=jnp.float32,
            )
            + b_ref[...]
        )

    out_ref[...] = x_ref[...] + te_ref[...][None]


def _fused_add(e, x, W, b, *, chunk=1024):
    B2, H, T, N = x.shape
    bh = B2 * H
    s = T // chunk
    xr = x.reshape(bh, T, N)
    out = pl.pallas_call(
        _fused_body,
        grid=(s, bh),
        in_specs=[
            pl.BlockSpec((chunk, N), lambda i, j: (i, 0)),
            pl.BlockSpec((N, N), lambda i, j: (0, 0)),
            pl.BlockSpec((1, N), lambda i, j: (0, 0)),
            pl.BlockSpec((1, chunk, N), lambda i, j: (j, i, 0)),
        ],
        out_specs=pl.BlockSpec((1, chunk, N), lambda i, j: (j, i, 0)),
        out_shape=jax.ShapeDtypeStruct((bh, T, N), jnp.float32),
        scratch_shapes=[pltpu.VMEM((chunk, N), jnp.float32)],
    )(e, W.astype(jnp.bfloat16), b.reshape(1, N), xr)
    return out.reshape(B2, H, T, N)


def kernel(x, t, emb, W, b):
    e = _sc_gather(emb, t)
    return _fused_add(e, x, W, b)


# speedup vs baseline: 1.1046x; 1.0198x over previous
import functools

import jax
import jax.numpy as jnp
from jax import lax
from jax.experimental import pallas as pl
from jax.experimental.pallas import tpu as pltpu
from jax.experimental.pallas import tpu_sc as plsc


def _sc_gather(emb, t):
    info = plsc.get_sparse_core_info()
    nw = info.num_cores * info.num_subcores
    B = t.shape[0]
    D = emb.shape[1]
    b_per_w = B // nw
    mesh = plsc.VectorSubcoreMesh(core_axis_name="c", subcore_axis_name="s")

    @functools.partial(
        pl.kernel,
        mesh=mesh,
        out_type=jax.ShapeDtypeStruct((B, D), jnp.float32),
        scratch_types=[
            pltpu.VMEM((b_per_w,), jnp.int32),
            pltpu.VMEM((b_per_w, D), jnp.float32),
            pltpu.SemaphoreType.DMA,
        ],
    )
    def gather(emb_hbm, t_hbm, out_hbm, idx_v, rows_v, sem):
        wid = lax.axis_index("s") * info.num_cores + lax.axis_index("c")
        base = wid * b_per_w
        pltpu.sync_copy(t_hbm.at[pl.ds(base, b_per_w)], idx_v)
        pltpu.async_copy(emb_hbm.at[idx_v], rows_v, sem).wait()
        pltpu.sync_copy(rows_v, out_hbm.at[pl.ds(base, b_per_w)])

    return gather(emb, t)


def _fused_body(e_ref, w_ref, b_ref, x_ref, out_ref, te_ref):
    @pl.when(pl.program_id(1) == 0)
    def _compute_te():
        te_ref[...] = (
            lax.dot_general(
                e_ref[...].astype(jnp.bfloat16), w_ref[...],
                (((1,), (1,)), ((), ())),
                preferred_element_type=jnp.float32,
            )
            + b_ref[...]
        )

    out_ref[...] = x_ref[...] + te_ref[...][None]


def _fused_add(e, x, W, b, *, chunk=1024, bhb=2):
    B2, H, T, N = x.shape
    bh = B2 * H
    s = T // chunk
    xr = x.reshape(bh, T, N)
    out = pl.pallas_call(
        _fused_body,
        grid=(s, bh // bhb),
        in_specs=[
            pl.BlockSpec((chunk, N), lambda i, j: (i, 0)),
            pl.BlockSpec((N, N), lambda i, j: (0, 0)),
            pl.BlockSpec((1, N), lambda i, j: (0, 0)),
            pl.BlockSpec((bhb, chunk, N), lambda i, j: (j, i, 0)),
        ],
        out_specs=pl.BlockSpec((bhb, chunk, N), lambda i, j: (j, i, 0)),
        out_shape=jax.ShapeDtypeStruct((bh, T, N), jnp.float32),
        scratch_shapes=[pltpu.VMEM((chunk, N), jnp.float32)],
    )(e, W.astype(jnp.bfloat16), b.reshape(1, N), xr)
    return out.reshape(B2, H, T, N)


def kernel(x, t, emb, W, b):
    e = _sc_gather(emb, t)
    return _fused_add(e, x, W, b)


# speedup vs baseline: 1.1142x; 1.0088x over previous
import functools

import jax
import jax.numpy as jnp
from jax import lax
from jax.experimental import pallas as pl
from jax.experimental.pallas import tpu as pltpu
from jax.experimental.pallas import tpu_sc as plsc


def _sc_gather(emb, t):
    info = plsc.get_sparse_core_info()
    nw = info.num_cores * info.num_subcores
    B = t.shape[0]
    D = emb.shape[1]
    b_per_w = B // nw
    mesh = plsc.VectorSubcoreMesh(core_axis_name="c", subcore_axis_name="s")

    @functools.partial(
        pl.kernel,
        mesh=mesh,
        out_type=jax.ShapeDtypeStruct((B, D), jnp.float32),
        scratch_types=[
            pltpu.VMEM((b_per_w,), jnp.int32),
            pltpu.VMEM((b_per_w, D), jnp.float32),
            pltpu.SemaphoreType.DMA,
        ],
    )
    def gather(emb_hbm, t_hbm, out_hbm, idx_v, rows_v, sem):
        wid = lax.axis_index("s") * info.num_cores + lax.axis_index("c")
        base = wid * b_per_w
        pltpu.sync_copy(t_hbm.at[pl.ds(base, b_per_w)], idx_v)
        pltpu.async_copy(emb_hbm.at[idx_v], rows_v, sem).wait()
        pltpu.sync_copy(rows_v, out_hbm.at[pl.ds(base, b_per_w)])

    return gather(emb, t)


def _fused_body(e_ref, w_ref, b_ref, x_ref, out_ref, te_ref):
    @pl.when(pl.program_id(1) == 0)
    def _compute_te():
        te_ref[...] = (
            lax.dot_general(
                e_ref[...].astype(jnp.bfloat16), w_ref[...],
                (((1,), (1,)), ((), ())),
                preferred_element_type=jnp.float32,
            )
            + b_ref[...]
        )

    out_ref[...] = x_ref[...] + te_ref[...][None]


def _fused_add(e, x, W, b, *, chunk=512, bhb=4):
    B2, H, T, N = x.shape
    bh = B2 * H
    s = T // chunk
    xr = x.reshape(bh, T, N)
    out = pl.pallas_call(
        _fused_body,
        grid=(s, bh // bhb),
        in_specs=[
            pl.BlockSpec((chunk, N), lambda i, j: (i, 0)),
            pl.BlockSpec((N, N), lambda i, j: (0, 0)),
            pl.BlockSpec((1, N), lambda i, j: (0, 0)),
            pl.BlockSpec((bhb, chunk, N), lambda i, j: (j, i, 0)),
        ],
        out_specs=pl.BlockSpec((bhb, chunk, N), lambda i, j: (j, i, 0)),
        out_shape=jax.ShapeDtypeStruct((bh, T, N), jnp.float32),
        scratch_shapes=[pltpu.VMEM((chunk, N), jnp.float32)],
    )(e, W.astype(jnp.bfloat16), b.reshape(1, N), xr)
    return out.reshape(B2, H, T, N)


def kernel(x, t, emb, W, b):
    e = _sc_gather(emb, t)
    return _fused_add(e, x, W, b)


# speedup vs baseline: 1.1222x; 1.0071x over previous
import functools

import jax
import jax.numpy as jnp
from jax import lax
from jax.experimental import pallas as pl
from jax.experimental.pallas import tpu as pltpu
from jax.experimental.pallas import tpu_sc as plsc


def _sc_gather(emb, t):
    info = plsc.get_sparse_core_info()
    nw = info.num_cores * info.num_subcores
    B = t.shape[0]
    D = emb.shape[1]
    b_per_w = B // nw
    mesh = plsc.VectorSubcoreMesh(core_axis_name="c", subcore_axis_name="s")

    @functools.partial(
        pl.kernel,
        mesh=mesh,
        out_type=jax.ShapeDtypeStruct((B, D), jnp.float32),
        scratch_types=[
            pltpu.VMEM((b_per_w,), jnp.int32),
            pltpu.VMEM((b_per_w, D), jnp.float32),
            pltpu.SemaphoreType.DMA,
        ],
    )
    def gather(emb_hbm, t_hbm, out_hbm, idx_v, rows_v, sem):
        wid = lax.axis_index("s") * info.num_cores + lax.axis_index("c")
        base = wid * b_per_w
        pltpu.sync_copy(t_hbm.at[pl.ds(base, b_per_w)], idx_v)
        pltpu.async_copy(emb_hbm.at[idx_v], rows_v, sem).wait()
        pltpu.sync_copy(rows_v, out_hbm.at[pl.ds(base, b_per_w)])

    return gather(emb, t)


def _fused_body(e_ref, w_ref, b_ref, x_ref, out_ref, te_ref):
    @pl.when(pl.program_id(1) == 0)
    def _compute_te():
        te_ref[...] = (
            lax.dot_general(
                e_ref[...].astype(jnp.bfloat16), w_ref[...],
                (((1,), (1,)), ((), ())),
                preferred_element_type=jnp.float32,
            )
            + b_ref[...]
        )

    out_ref[...] = x_ref[...] + te_ref[...][None]


def _fused_add(e, x, W, b, *, chunk=256, bhb=8):
    B2, H, T, N = x.shape
    bh = B2 * H
    s = T // chunk
    xr = x.reshape(bh, T, N)
    out = pl.pallas_call(
        _fused_body,
        grid=(s, bh // bhb),
        in_specs=[
            pl.BlockSpec((chunk, N), lambda i, j: (i, 0)),
            pl.BlockSpec((N, N), lambda i, j: (0, 0)),
            pl.BlockSpec((1, N), lambda i, j: (0, 0)),
            pl.BlockSpec((bhb, chunk, N), lambda i, j: (j, i, 0)),
        ],
        out_specs=pl.BlockSpec((bhb, chunk, N), lambda i, j: (j, i, 0)),
        out_shape=jax.ShapeDtypeStruct((bh, T, N), jnp.float32),
        scratch_shapes=[pltpu.VMEM((chunk, N), jnp.float32)],
    )(e, W.astype(jnp.bfloat16), b.reshape(1, N), xr)
    return out.reshape(B2, H, T, N)


def kernel(x, t, emb, W, b):
    e = _sc_gather(emb, t)
    return _fused_add(e, x, W, b)


# speedup vs baseline: 1.1237x; 1.0014x over previous
import functools

import jax
import jax.numpy as jnp
from jax import lax
from jax.experimental import pallas as pl
from jax.experimental.pallas import tpu as pltpu
from jax.experimental.pallas import tpu_sc as plsc


def _sc_gather(emb, t):
    info = plsc.get_sparse_core_info()
    nw = info.num_cores * info.num_subcores
    B = t.shape[0]
    D = emb.shape[1]
    b_per_w = B // nw
    mesh = plsc.VectorSubcoreMesh(core_axis_name="c", subcore_axis_name="s")

    @functools.partial(
        pl.kernel,
        mesh=mesh,
        out_type=jax.ShapeDtypeStruct((B, D), jnp.float32),
        scratch_types=[
            pltpu.VMEM((b_per_w,), jnp.int32),
            pltpu.VMEM((b_per_w, D), jnp.float32),
            pltpu.SemaphoreType.DMA,
        ],
    )
    def gather(emb_hbm, t_hbm, out_hbm, idx_v, rows_v, sem):
        wid = lax.axis_index("s") * info.num_cores + lax.axis_index("c")
        base = wid * b_per_w
        pltpu.sync_copy(t_hbm.at[pl.ds(base, b_per_w)], idx_v)
        pltpu.async_copy(emb_hbm.at[idx_v], rows_v, sem).wait()
        pltpu.sync_copy(rows_v, out_hbm.at[pl.ds(base, b_per_w)])

    return gather(emb, t)


def _fused_body(e_ref, w_ref, b_ref, x_ref, out_ref, te_ref):
    @pl.when(pl.program_id(1) == 0)
    def _compute_te():
        te_ref[...] = (
            lax.dot_general(
                e_ref[...].astype(jnp.bfloat16), w_ref[...],
                (((1,), (1,)), ((), ())),
                preferred_element_type=jnp.float32,
            )
            + b_ref[...]
        )

    out_ref[...] = x_ref[...] + te_ref[...][None]


def _fused_add(e, x, W, b, *, chunk=128, bhb=16):
    B2, H, T, N = x.shape
    bh = B2 * H
    s = T // chunk
    xr = x.reshape(bh, T, N)
    out = pl.pallas_call(
        _fused_body,
        grid=(s, bh // bhb),
        in_specs=[
            pl.BlockSpec((chunk, N), lambda i, j: (i, 0)),
            pl.BlockSpec((N, N), lambda i, j: (0, 0)),
            pl.BlockSpec((1, N), lambda i, j: (0, 0)),
            pl.BlockSpec((bhb, chunk, N), lambda i, j: (j, i, 0)),
        ],
        out_specs=pl.BlockSpec((bhb, chunk, N), lambda i, j: (j, i, 0)),
        out_shape=jax.ShapeDtypeStruct((bh, T, N), jnp.float32),
        scratch_shapes=[pltpu.VMEM((chunk, N), jnp.float32)],
    )(e, W.astype(jnp.bfloat16), b.reshape(1, N), xr)
    return out.reshape(B2, H, T, N)


def kernel(x, t, emb, W, b):
    e = _sc_gather(emb, t)
    return _fused_add(e, x, W, b)
